# baseline (speedup 1.0000x reference)
import functools

import jax
import jax.numpy as jnp
from jax import lax
from jax.experimental import pallas as pl
from jax.experimental.pallas import tpu as pltpu
from jax.experimental.pallas import tpu_sc as plsc

D = 32
B = 16384
NC, NS, L = 2, 16, 16
NW = NC * NS
BPW = B // NW
G = BPW // L

_mesh = plsc.VectorSubcoreMesh(
    core_axis_name="c", subcore_axis_name="s", num_cores=NC, num_subcores=NS
)


@functools.partial(
    pl.kernel,
    out_type=jax.ShapeDtypeStruct((B,), jnp.float32),
    mesh=_mesh,
    compiler_params=pltpu.CompilerParams(
        needs_layout_passes=False, use_tc_tiling_on_sc=False
    ),
    scratch_types=[
        pltpu.VMEM((BPW,), jnp.int32),
        pltpu.VMEM((BPW, D), jnp.float32),
        pltpu.VMEM((D, L), jnp.float32),
        pltpu.VMEM((BPW,), jnp.float32),
        pltpu.SemaphoreType.DMA,
    ],
)
def _sc_kernel(idx_hbm, userb_hbm, table_hbm, out_hbm, idx_v, rows_v, ub_v,
               out_v, sem):
    wid = lax.axis_index("s") * NC + lax.axis_index("c")
    base = wid * BPW
    pltpu.sync_copy(idx_hbm.at[pl.ds(base, BPW)], idx_v)
    pltpu.sync_copy(userb_hbm, ub_v)
    pltpu.async_copy(table_hbm.at[idx_v], rows_v, sem).wait()
    lane = lax.iota(jnp.int32, L)

    def group(g, carry):
        row_idx = g * L + lane
        acc = jnp.zeros((L,), jnp.float32)
        for j in range(D):
            col = plsc.load_gather(
                rows_v, [row_idx, jnp.full((L,), j, jnp.int32)]
            )
            acc = acc + col * ub_v[j]
        out_v[pl.ds(g * L, L)] = 1.0 / (1.0 + jnp.exp(-acc))
        return carry

    lax.fori_loop(0, G, group, 0)
    pltpu.sync_copy(out_v, out_hbm.at[pl.ds(base, BPW)])


def kernel(item_indices, embedding_user, embedding_item):
    userb = jnp.broadcast_to(embedding_user.reshape(D, 1), (D, L))
    return _sc_kernel(item_indices, userb, embedding_item)


# speedup vs baseline: 7.5985x; 7.5985x over previous
import functools

import jax
import jax.numpy as jnp
from jax import lax
from jax.experimental import pallas as pl
from jax.experimental.pallas import tpu as pltpu
from jax.experimental.pallas import tpu_sc as plsc

D = 32
B = 16384
N = 1000000
NPAD = 1000064
ROWS = NPAD // 128
CHUNK = 32768
NCHUNK = -(-NPAD // CHUNK)
NC, NS, L = 2, 16, 16
NW = NC * NS
BPW = B // NW
G = BPW // L


def _dot_body(u_ref, t_ref, out_ref):
    x = t_ref[...]
    u = u_ref[...]
    y = jax.lax.dot_general(
        u, x, (((1,), (0,)), ((), ())),
        preferred_element_type=jnp.float32,
    )
    out_ref[...] = y[0]


_dot_call = pl.pallas_call(
    _dot_body,
    grid=(NCHUNK,),
    in_specs=[
        pl.BlockSpec((8, D), lambda i: (0, 0)),
        pl.BlockSpec((D, CHUNK), lambda i: (0, i)),
    ],
    out_specs=pl.BlockSpec((CHUNK,), lambda i: (i,)),
    out_shape=jax.ShapeDtypeStruct((NPAD,), jnp.float32),
)

_mesh = plsc.VectorSubcoreMesh(
    core_axis_name="c", subcore_axis_name="s", num_cores=NC, num_subcores=NS
)


@functools.partial(
    pl.kernel,
    out_type=jax.ShapeDtypeStruct((B,), jnp.float32),
    mesh=_mesh,
    compiler_params=pltpu.CompilerParams(needs_layout_passes=False),
    scratch_types=[
        pltpu.VMEM((BPW,), jnp.int32),
        pltpu.VMEM((BPW,), jnp.int32),
        pltpu.VMEM((BPW, 128), jnp.float32),
        pltpu.VMEM((BPW,), jnp.float32),
        pltpu.SemaphoreType.DMA,
    ],
)
def _sc_kernel(idx_hbm, logits_hbm, out_hbm, idx_v, row_v, rows_v, out_v,
               sem):
    wid = lax.axis_index("s") * NC + lax.axis_index("c")
    base = wid * BPW
    pltpu.sync_copy(idx_hbm.at[pl.ds(base, BPW)], idx_v)

    def split(g, carry):
        v = idx_v[pl.ds(g * L, L)]
        row_v[pl.ds(g * L, L)] = lax.shift_right_logical(v, 7)
        return carry

    lax.fori_loop(0, G, split, 0)
    pltpu.async_copy(logits_hbm.at[row_v], rows_v, sem).wait()
    lane = lax.iota(jnp.int32, L)

    def group(g, carry):
        col = jnp.bitwise_and(idx_v[pl.ds(g * L, L)], 127)
        x = plsc.load_gather(rows_v, [g * L + lane, col])
        out_v[pl.ds(g * L, L)] = 1.0 / (1.0 + jnp.exp(-x))
        return carry

    lax.fori_loop(0, G, group, 0)
    pltpu.sync_copy(out_v, out_hbm.at[pl.ds(base, BPW)])


def kernel(item_indices, embedding_user, embedding_item):
    u8 = jnp.broadcast_to(embedding_user.reshape(1, D), (8, D))
    logits = _dot_call(u8, embedding_item.T).reshape(ROWS, 128)
    return _sc_kernel(item_indices, logits)


# speedup vs baseline: 7.8327x; 1.0308x over previous
import functools

import jax
import jax.numpy as jnp
from jax import lax
from jax.experimental import pallas as pl
from jax.experimental.pallas import tpu as pltpu
from jax.experimental.pallas import tpu_sc as plsc

D = 32
B = 16384
N = 1000000
NPAD = 1000064
ROWS = NPAD // 128
CHUNK = 131072
NCHUNK = -(-NPAD // CHUNK)
NC, NS, L = 2, 16, 16
NW = NC * NS
BPW = B // NW
G = BPW // L


def _dot_body(u_ref, t_ref, out_ref):
    x = t_ref[...]
    u = u_ref[...]
    y = jax.lax.dot_general(
        u, x, (((1,), (0,)), ((), ())),
        preferred_element_type=jnp.float32,
    )
    out_ref[...] = y[0]


_dot_call = pl.pallas_call(
    _dot_body,
    grid=(NCHUNK,),
    in_specs=[
        pl.BlockSpec((8, D), lambda i: (0, 0)),
        pl.BlockSpec((D, CHUNK), lambda i: (0, i)),
    ],
    out_specs=pl.BlockSpec((CHUNK,), lambda i: (i,)),
    out_shape=jax.ShapeDtypeStruct((NPAD,), jnp.float32),
)

_mesh = plsc.VectorSubcoreMesh(
    core_axis_name="c", subcore_axis_name="s", num_cores=NC, num_subcores=NS
)


@functools.partial(
    pl.kernel,
    out_type=jax.ShapeDtypeStruct((B,), jnp.float32),
    mesh=_mesh,
    compiler_params=pltpu.CompilerParams(needs_layout_passes=False),
    scratch_types=[
        pltpu.VMEM((BPW,), jnp.int32),
        pltpu.VMEM((BPW,), jnp.int32),
        pltpu.VMEM((BPW, 128), jnp.float32),
        pltpu.VMEM((BPW,), jnp.float32),
        pltpu.SemaphoreType.DMA,
    ],
)
def _sc_kernel(idx_hbm, logits_hbm, out_hbm, idx_v, row_v, rows_v, out_v,
               sem):
    wid = lax.axis_index("s") * NC + lax.axis_index("c")
    base = wid * BPW
    pltpu.sync_copy(idx_hbm.at[pl.ds(base, BPW)], idx_v)

    def split(g, carry):
        v = idx_v[pl.ds(g * L, L)]
        row_v[pl.ds(g * L, L)] = lax.shift_right_logical(v, 7)
        return carry

    lax.fori_loop(0, G, split, 0)
    pltpu.async_copy(logits_hbm.at[row_v], rows_v, sem).wait()
    lane = lax.iota(jnp.int32, L)

    def group(g, carry):
        col = jnp.bitwise_and(idx_v[pl.ds(g * L, L)], 127)
        x = plsc.load_gather(rows_v, [g * L + lane, col])
        out_v[pl.ds(g * L, L)] = 1.0 / (1.0 + jnp.exp(-x))
        return carry

    lax.fori_loop(0, G, group, 0)
    pltpu.sync_copy(out_v, out_hbm.at[pl.ds(base, BPW)])


def kernel(item_indices, embedding_user, embedding_item):
    u8 = jnp.broadcast_to(embedding_user.reshape(1, D), (8, D))
    logits = _dot_call(u8, embedding_item.T).reshape(ROWS, 128)
    return _sc_kernel(item_indices, logits)
